# baseline (speedup 1.0000x reference)
import functools

import jax
import jax.numpy as jnp
from jax import lax
from jax.experimental import pallas as pl
from jax.experimental.pallas import tpu as pltpu
from jax.experimental.pallas import tpu_sc as plsc

_F = 26
_B = 16384
_EMBED = 32
_V = 1000000
_VB = (_V + 127) // 128
_NQ = _VB * 32
_BC = 128
_W = 512
_SUB = _W // _BC
_UNITS = _F * _SUB
_NIDX = _F * _W
_TB = 245


def _lookup(idx_t, tab_t):
    info = plsc.get_sparse_core_info()
    nw = info.num_cores * info.num_subcores
    assert nw * _W == _B

    mesh = plsc.VectorSubcoreMesh(core_axis_name="c", subcore_axis_name="s")

    @functools.partial(
        pl.kernel,
        mesh=mesh,
        out_type=(
            jax.ShapeDtypeStruct((_F, _EMBED, _B), jnp.float32),
            jax.ShapeDtypeStruct((_NQ, 128), jnp.float32),
        ),
        scratch_types=[
            pltpu.VMEM((_NIDX,), jnp.int32),
            pltpu.VMEM((_NIDX,), jnp.int32),
            pltpu.VMEM((4 * _BC, 128), jnp.float32),
            pltpu.VMEM((2 * _EMBED, _BC), jnp.float32),
            pltpu.VMEM((2 * _EMBED, 128), jnp.float32),
            pltpu.VMEM((2 * _EMBED, 128), jnp.float32),
            pltpu.SemaphoreType.DMA,
            pltpu.SemaphoreType.DMA,
            pltpu.SemaphoreType.DMA,
            pltpu.SemaphoreType.DMA,
            pltpu.SemaphoreType.DMA,
            pltpu.SemaphoreType.REGULAR,
        ],
        compiler_params=pltpu.CompilerParams(
            use_tc_tiling_on_sc=True, needs_layout_passes=False
        ),
    )
    def k(idx_hbm, tab_hbm, out_hbm, scr_hbm, idxb, gidx, rows4, outb,
          tbuf, qbuf, isem, gsem, wsem, tsem, osem, barsem):
        wid = lax.axis_index("s") * info.num_cores + lax.axis_index("c")
        b0w = wid * _W

        icps = [
            pltpu.async_copy(
                idx_hbm.at[f, pl.ds(b0w, _W)],
                idxb.at[pl.ds(f * _W, _W)],
                isem,
            )
            for f in range(_F)
        ]

        def vblock(c):
            return jnp.minimum(c * nw + wid, _VB - 1)

        def fire_slab(c, slot):
            return pltpu.async_copy(
                tab_hbm.at[:, pl.ds(vblock(c) * 128, 128)],
                tbuf.at[pl.ds(slot * _EMBED, _EMBED), :],
                tsem,
            )

        fire_slab(0, 0)

        iot = jnp.arange(16, dtype=jnp.int32)

        def tunit(c, kk):
            pltpu.make_async_copy(
                tab_hbm.at[:, pl.ds(0, 128)],
                tbuf.at[pl.ds(0, _EMBED), :],
                tsem,
            ).wait()

            @pl.when(c + 1 < _TB)
            def _():
                fire_slab(c + 1, (kk + 1) % 2)

            @pl.when(c >= 2)
            def _():
                pltpu.make_async_copy(
                    qbuf.at[pl.ds(0, _EMBED), :],
                    scr_hbm.at[pl.ds(0, _EMBED), :],
                    osem,
                ).wait()

            trow = kk * _EMBED
            for q in range(32):
                for j in range(8):
                    rowvec = trow + 16 * (j % 2) + iot
                    colvec = jnp.full((16,), q * 4 + j // 2, jnp.int32)
                    qbuf.at[trow + q][pl.ds(j * 16, 16)] = (
                        plsc.load_gather(tbuf, [rowvec, colvec])
                    )
            pltpu.async_copy(
                qbuf.at[pl.ds(trow, _EMBED), :],
                scr_hbm.at[pl.ds(vblock(c) * 32, 32), :],
                osem,
            )

        def tbody(cc, carry):
            for kk in range(2):
                tunit(cc * 2 + kk, kk)
            return carry

        lax.fori_loop(0, _TB // 2, tbody, 0)
        tunit(_TB - 1, 0)
        for _ in range(2):
            pltpu.make_async_copy(
                qbuf.at[pl.ds(0, _EMBED), :],
                scr_hbm.at[pl.ds(0, _EMBED), :],
                osem,
            ).wait()

        plsc.subcore_barrier()
        pltpu.core_barrier(barsem, core_axis_name="c")
        plsc.subcore_barrier()

        for cp in icps:
            cp.wait()

        def prep(i, carry):
            v = idxb[pl.ds(i * 16, 16)]
            gidx[pl.ds(i * 16, 16)] = lax.shift_right_logical(v, 2)
            return carry

        lax.fori_loop(0, _NIDX // 16, prep, 0)

        def unit_off(c):
            f = lax.rem(c, _F)
            sub = lax.div(c, _F)
            return f, sub, f * _W + sub * _BC

        def fire_gather(c, slot):
            _, _, off = unit_off(c)
            return pltpu.async_copy(
                scr_hbm.at[gidx.at[pl.ds(off, _BC)]],
                rows4.at[pl.ds(slot * _BC, _BC), :],
                gsem,
            )

        fire_gather(0, 0)
        fire_gather(1, 1)

        def body(cc, carry):
            for kk in range(4):
                c = cc * 4 + kk
                f, sub, off = unit_off(c)

                @pl.when(c + 2 < _UNITS)
                def _():
                    fire_gather(c + 2, (kk + 2) % 4)

                pltpu.make_async_copy(
                    scr_hbm.at[gidx.at[pl.ds(0, _BC)]],
                    rows4.at[pl.ds(0, _BC), :],
                    gsem,
                ).wait()

                @pl.when(c >= 2)
                def _():
                    pltpu.make_async_copy(
                        outb.at[pl.ds(0, _EMBED), :],
                        out_hbm.at[0, :, pl.ds(0, _BC)],
                        wsem,
                    ).wait()

                orow = (kk % 2) * _EMBED
                for j in range(_BC // 16):
                    rowvec = iot + (kk * _BC + j * 16)
                    lb = (idxb[pl.ds(off + j * 16, 16)] & 3) * _EMBED
                    for e in range(_EMBED):
                        outb.at[orow + e][pl.ds(j * 16, 16)] = (
                            plsc.load_gather(rows4, [rowvec, lb + e])
                        )
                pltpu.async_copy(
                    outb.at[pl.ds(orow, _EMBED), :],
                    out_hbm.at[f, :, pl.ds(b0w + sub * _BC, _BC)],
                    wsem,
                )
            return carry

        lax.fori_loop(0, _UNITS // 4, body, 0)
        for _ in range(2):
            pltpu.make_async_copy(
                outb.at[pl.ds(0, _EMBED), :],
                out_hbm.at[0, :, pl.ds(0, _BC)],
                wsem,
            ).wait()

    return k(idx_t, tab_t)


def kernel(tensor, table):
    idx_t = tensor.T.astype(jnp.int32)
    tab_t = table.T
    out_t, _ = _lookup(idx_t, tab_t)
    return out_t.transpose(2, 0, 1)


# speedup vs baseline: 1.4051x; 1.4051x over previous
import functools

import jax
import jax.numpy as jnp
from jax import lax
from jax.experimental import pallas as pl
from jax.experimental.pallas import tpu as pltpu
from jax.experimental.pallas import tpu_sc as plsc

_F = 26
_B = 16384
_EMBED = 32
_BC = 128
_W = 512
_SUB = _W // _BC
_UNITS = _F * _SUB
_NIDX = _F * _W


def _lookup(idx_t, tabp):
    info = plsc.get_sparse_core_info()
    nw = info.num_cores * info.num_subcores
    assert nw * _W == _B

    mesh = plsc.VectorSubcoreMesh(core_axis_name="c", subcore_axis_name="s")

    @functools.partial(
        pl.kernel,
        mesh=mesh,
        out_type=jax.ShapeDtypeStruct((_F, _EMBED, _B), jnp.float32),
        scratch_types=[
            pltpu.VMEM((_NIDX,), jnp.int32),
            pltpu.VMEM((4 * _BC, 128), jnp.float32),
            pltpu.VMEM((2 * _EMBED, _BC), jnp.float32),
            pltpu.SemaphoreType.DMA,
            pltpu.SemaphoreType.DMA,
            pltpu.SemaphoreType.DMA,
        ],
        compiler_params=pltpu.CompilerParams(
            use_tc_tiling_on_sc=True, needs_layout_passes=False
        ),
    )
    def k(idx_hbm, tab_hbm, out_hbm, idxb, rows, outb, isem, gsem, wsem):
        wid = lax.axis_index("s") * info.num_cores + lax.axis_index("c")
        b0w = wid * _W

        icps = [
            pltpu.async_copy(
                idx_hbm.at[f, pl.ds(b0w, _W)],
                idxb.at[pl.ds(f * _W, _W)],
                isem,
            )
            for f in range(_F)
        ]
        for cp in icps:
            cp.wait()

        def unit_off(c):
            f = lax.rem(c, _F)
            sub = lax.div(c, _F)
            return f, sub, f * _W + sub * _BC

        def fire_gather(c, slot):
            _, _, off = unit_off(c)
            return pltpu.async_copy(
                tab_hbm.at[idxb.at[pl.ds(off, _BC)]],
                rows.at[pl.ds(slot * _BC, _BC), :],
                gsem,
            )

        fire_gather(0, 0)
        fire_gather(1, 1)

        iot = jnp.arange(16, dtype=jnp.int32)

        def body(cc, carry):
            for kk in range(4):
                c = cc * 4 + kk
                f, sub, off = unit_off(c)

                @pl.when(c + 2 < _UNITS)
                def _():
                    fire_gather(c + 2, (kk + 2) % 4)

                pltpu.make_async_copy(
                    tab_hbm.at[idxb.at[pl.ds(0, _BC)]],
                    rows.at[pl.ds(0, _BC), :],
                    gsem,
                ).wait()

                @pl.when(c >= 2)
                def _():
                    pltpu.make_async_copy(
                        outb.at[pl.ds(0, _EMBED), :],
                        out_hbm.at[0, :, pl.ds(0, _BC)],
                        wsem,
                    ).wait()

                orow = (kk % 2) * _EMBED
                for j in range(_BC // 16):
                    rowvec = iot + (kk * _BC + j * 16)
                    for e in range(_EMBED):
                        colvec = jnp.full((16,), e, jnp.int32)
                        outb.at[orow + e][pl.ds(j * 16, 16)] = (
                            plsc.load_gather(rows, [rowvec, colvec])
                        )
                pltpu.async_copy(
                    outb.at[pl.ds(orow, _EMBED), :],
                    out_hbm.at[f, :, pl.ds(b0w + sub * _BC, _BC)],
                    wsem,
                )
            return carry

        lax.fori_loop(0, _UNITS // 4, body, 0)
        for _ in range(2):
            pltpu.make_async_copy(
                outb.at[pl.ds(0, _EMBED), :],
                out_hbm.at[0, :, pl.ds(0, _BC)],
                wsem,
            ).wait()

    return k(idx_t, tabp)


def kernel(tensor, table):
    idx_t = tensor.T.astype(jnp.int32)
    tabp = jnp.pad(table, ((0, 0), (0, 96)))
    out_t = _lookup(idx_t, tabp)
    return out_t.transpose(2, 0, 1)


# speedup vs baseline: 1.5240x; 1.0846x over previous
import functools

import jax
import jax.numpy as jnp
from jax import lax
from jax.experimental import pallas as pl
from jax.experimental.pallas import tpu as pltpu
from jax.experimental.pallas import tpu_sc as plsc

_F = 26
_B = 16384
_EMBED = 32
_V = 1000000
_VB = (_V + 127) // 128
_NQ = _VB * 32
_BC = 128
_W = 512
_SUB = _W // _BC
_UNITS = _F * _SUB
_NIDX = _F * _W
_TB = 245


def _lookup(idx_t, tab_t):
    info = plsc.get_sparse_core_info()
    nw = info.num_cores * info.num_subcores
    assert nw * _W == _B

    mesh = plsc.VectorSubcoreMesh(core_axis_name="c", subcore_axis_name="s")

    @functools.partial(
        pl.kernel,
        mesh=mesh,
        out_type=(
            jax.ShapeDtypeStruct((_F, _EMBED, _B), jnp.float32),
            jax.ShapeDtypeStruct((_NQ, 128), jnp.float32),
        ),
        scratch_types=[
            pltpu.VMEM((_NIDX,), jnp.int32),
            pltpu.VMEM((_NIDX,), jnp.int32),
            pltpu.VMEM((4 * _BC, 128), jnp.float32),
            pltpu.VMEM((2 * _EMBED, _BC), jnp.float32),
            pltpu.VMEM((2 * _EMBED, 128), jnp.float32),
            pltpu.VMEM((2 * _EMBED, 128), jnp.float32),
            pltpu.SemaphoreType.DMA,
            pltpu.SemaphoreType.DMA,
            pltpu.SemaphoreType.DMA,
            pltpu.SemaphoreType.DMA,
            pltpu.SemaphoreType.DMA,
            pltpu.SemaphoreType.REGULAR,
        ],
        compiler_params=pltpu.CompilerParams(
            use_tc_tiling_on_sc=True, needs_layout_passes=False
        ),
    )
    def k(idx_hbm, tab_hbm, out_hbm, scr_hbm, idxb, gidx, rows4, outb,
          tbuf, qbuf, isem, gsem, wsem, tsem, osem, barsem):
        wid = lax.axis_index("s") * info.num_cores + lax.axis_index("c")
        b0w = wid * _W

        icps = [
            pltpu.async_copy(
                idx_hbm.at[f, pl.ds(b0w, _W)],
                idxb.at[pl.ds(f * _W, _W)],
                isem,
            )
            for f in range(_F)
        ]

        def vblock(c):
            return jnp.minimum(c * nw + wid, _VB - 1)

        def fire_slab(c, slot):
            return pltpu.async_copy(
                tab_hbm.at[:, pl.ds(vblock(c) * 128, 128)],
                tbuf.at[pl.ds(slot * _EMBED, _EMBED), :],
                tsem,
            )

        fire_slab(0, 0)

        iot = jnp.arange(16, dtype=jnp.int32)

        def tunit(c, kk):
            pltpu.make_async_copy(
                tab_hbm.at[:, pl.ds(0, 128)],
                tbuf.at[pl.ds(0, _EMBED), :],
                tsem,
            ).wait()

            @pl.when(c + 1 < _TB)
            def _():
                fire_slab(c + 1, (kk + 1) % 2)

            @pl.when(c >= 2)
            def _():
                pltpu.make_async_copy(
                    qbuf.at[pl.ds(0, _EMBED), :],
                    scr_hbm.at[pl.ds(0, _EMBED), :],
                    osem,
                ).wait()

            trow = kk * _EMBED
            for q in range(32):
                vals = [
                    plsc.load_gather(
                        tbuf,
                        [
                            trow + 16 * (j % 2) + iot,
                            jnp.full((16,), q * 4 + j // 2, jnp.int32),
                        ],
                    )
                    for j in range(8)
                ]
                for j in range(8):
                    qbuf.at[trow + q][pl.ds(j * 16, 16)] = vals[j]
            pltpu.async_copy(
                qbuf.at[pl.ds(trow, _EMBED), :],
                scr_hbm.at[pl.ds(vblock(c) * 32, 32), :],
                osem,
            )

        def tbody(cc, carry):
            for kk in range(2):
                tunit(cc * 2 + kk, kk)
            return carry

        lax.fori_loop(0, _TB // 2, tbody, 0)
        tunit(_TB - 1, 0)
        for _ in range(2):
            pltpu.make_async_copy(
                qbuf.at[pl.ds(0, _EMBED), :],
                scr_hbm.at[pl.ds(0, _EMBED), :],
                osem,
            ).wait()

        plsc.subcore_barrier()
        pltpu.core_barrier(barsem, core_axis_name="c")
        plsc.subcore_barrier()

        for cp in icps:
            cp.wait()

        def prep(i, carry):
            v = idxb[pl.ds(i * 16, 16)]
            gidx[pl.ds(i * 16, 16)] = lax.shift_right_logical(v, 2)
            return carry

        lax.fori_loop(0, _NIDX // 16, prep, 0)

        def unit_off(c):
            f = lax.rem(c, _F)
            sub = lax.div(c, _F)
            return f, sub, f * _W + sub * _BC

        def fire_gather(c, slot):
            _, _, off = unit_off(c)
            return pltpu.async_copy(
                scr_hbm.at[gidx.at[pl.ds(off, _BC)]],
                rows4.at[pl.ds(slot * _BC, _BC), :],
                gsem,
            )

        fire_gather(0, 0)
        fire_gather(1, 1)

        def body(cc, carry):
            for kk in range(4):
                c = cc * 4 + kk
                f, sub, off = unit_off(c)

                @pl.when(c + 2 < _UNITS)
                def _():
                    fire_gather(c + 2, (kk + 2) % 4)

                pltpu.make_async_copy(
                    scr_hbm.at[gidx.at[pl.ds(0, _BC)]],
                    rows4.at[pl.ds(0, _BC), :],
                    gsem,
                ).wait()

                @pl.when(c >= 2)
                def _():
                    pltpu.make_async_copy(
                        outb.at[pl.ds(0, _EMBED), :],
                        out_hbm.at[0, :, pl.ds(0, _BC)],
                        wsem,
                    ).wait()

                orow = (kk % 2) * _EMBED
                for j in range(_BC // 16):
                    rowvec = iot + (kk * _BC + j * 16)
                    lb = (idxb[pl.ds(off + j * 16, 16)] & 3) * _EMBED
                    for eb in range(_EMBED // 8):
                        vals = [
                            plsc.load_gather(rows4, [rowvec, lb + (eb * 8 + i)])
                            for i in range(8)
                        ]
                        for i in range(8):
                            outb.at[orow + eb * 8 + i][pl.ds(j * 16, 16)] = (
                                vals[i]
                            )
                pltpu.async_copy(
                    outb.at[pl.ds(orow, _EMBED), :],
                    out_hbm.at[f, :, pl.ds(b0w + sub * _BC, _BC)],
                    wsem,
                )
            return carry

        lax.fori_loop(0, _UNITS // 4, body, 0)
        for _ in range(2):
            pltpu.make_async_copy(
                outb.at[pl.ds(0, _EMBED), :],
                out_hbm.at[0, :, pl.ds(0, _BC)],
                wsem,
            ).wait()

    return k(idx_t, tab_t)


def kernel(tensor, table):
    idx_t = tensor.T.astype(jnp.int32)
    tab_t = table.T
    out_t, _ = _lookup(idx_t, tab_t)
    return out_t.transpose(2, 0, 1)


# speedup vs baseline: 1.5673x; 1.0284x over previous
import functools

import jax
import jax.numpy as jnp
from jax import lax
from jax.experimental import pallas as pl
from jax.experimental.pallas import tpu as pltpu
from jax.experimental.pallas import tpu_sc as plsc

_F = 26
_B = 16384
_EMBED = 32
_V = 1000000
_VB = (_V + 127) // 128
_NQ = _VB * 32
_BC = 128
_W = 512
_SUB = _W // _BC
_UNITS = _F * _SUB
_NIDX = _F * _W
_TB = 245


def _lookup(idx_t, tab_t):
    info = plsc.get_sparse_core_info()
    nw = info.num_cores * info.num_subcores
    assert nw * _W == _B

    mesh = plsc.VectorSubcoreMesh(core_axis_name="c", subcore_axis_name="s")

    @functools.partial(
        pl.kernel,
        mesh=mesh,
        out_type=(
            jax.ShapeDtypeStruct((_F, _EMBED, _B), jnp.float32),
            jax.ShapeDtypeStruct((_NQ, 128), jnp.float32),
        ),
        scratch_types=[
            pltpu.VMEM((_NIDX,), jnp.int32),
            pltpu.VMEM((_NIDX,), jnp.int32),
            pltpu.VMEM((4 * _BC, 128), jnp.float32),
            pltpu.VMEM((2 * _EMBED, _BC), jnp.float32),
            pltpu.VMEM((2 * _EMBED, 128), jnp.float32),
            pltpu.VMEM((2 * _EMBED, 128), jnp.float32),
            pltpu.SemaphoreType.DMA,
            pltpu.SemaphoreType.DMA,
            pltpu.SemaphoreType.DMA,
            pltpu.SemaphoreType.DMA,
            pltpu.SemaphoreType.DMA,
            pltpu.SemaphoreType.REGULAR,
        ],
        compiler_params=pltpu.CompilerParams(
            use_tc_tiling_on_sc=True, needs_layout_passes=False
        ),
    )
    def k(idx_hbm, tab_hbm, out_hbm, scr_hbm, idxb, gidx, rows4, outb,
          tbuf, qbuf, isem, gsem, wsem, tsem, osem, barsem):
        wid = lax.axis_index("s") * info.num_cores + lax.axis_index("c")
        b0w = wid * _W

        icps = [
            pltpu.async_copy(
                idx_hbm.at[f, pl.ds(b0w, _W)],
                idxb.at[pl.ds(f * _W, _W)],
                isem,
            )
            for f in range(_F)
        ]

        def vblock(c):
            return jnp.minimum(c * nw + wid, _VB - 1)

        def fire_slab(c, slot):
            return pltpu.async_copy(
                tab_hbm.at[:, pl.ds(vblock(c) * 128, 128)],
                tbuf.at[pl.ds(slot * _EMBED, _EMBED), :],
                tsem,
            )

        fire_slab(0, 0)

        iot = jnp.arange(16, dtype=jnp.int32)

        def tunit(c, kk):
            pltpu.make_async_copy(
                tab_hbm.at[:, pl.ds(0, 128)],
                tbuf.at[pl.ds(0, _EMBED), :],
                tsem,
            ).wait()

            @pl.when(c + 1 < _TB)
            def _():
                fire_slab(c + 1, (kk + 1) % 2)

            @pl.when(c >= 2)
            def _():
                pltpu.make_async_copy(
                    qbuf.at[pl.ds(0, _EMBED), :],
                    scr_hbm.at[pl.ds(0, _EMBED), :],
                    osem,
                ).wait()

            trow = kk * _EMBED
            for qq in range(16):
                vals = [
                    plsc.load_gather(
                        tbuf,
                        [
                            trow + 16 * (j % 2) + iot,
                            jnp.full(
                                (16,), (qq * 2 + j // 8) * 4 + (j // 2) % 4,
                                jnp.int32,
                            ),
                        ],
                    )
                    for j in range(16)
                ]
                for j in range(16):
                    qbuf.at[trow + qq * 2 + j // 8][pl.ds((j % 8) * 16, 16)] = (
                        vals[j]
                    )
            pltpu.async_copy(
                qbuf.at[pl.ds(trow, _EMBED), :],
                scr_hbm.at[pl.ds(vblock(c) * 32, 32), :],
                osem,
            )

        def tbody(cc, carry):
            for kk in range(2):
                tunit(cc * 2 + kk, kk)
            return carry

        lax.fori_loop(0, _TB // 2, tbody, 0)
        tunit(_TB - 1, 0)
        for _ in range(2):
            pltpu.make_async_copy(
                qbuf.at[pl.ds(0, _EMBED), :],
                scr_hbm.at[pl.ds(0, _EMBED), :],
                osem,
            ).wait()

        plsc.subcore_barrier()
        pltpu.core_barrier(barsem, core_axis_name="c")
        plsc.subcore_barrier()

        for cp in icps:
            cp.wait()

        def prep(i, carry):
            v = idxb[pl.ds(i * 16, 16)]
            gidx[pl.ds(i * 16, 16)] = lax.shift_right_logical(v, 2)
            return carry

        lax.fori_loop(0, _NIDX // 16, prep, 0)

        def unit_off(c):
            f = lax.rem(c, _F)
            sub = lax.div(c, _F)
            return f, sub, f * _W + sub * _BC

        def fire_gather(c, slot):
            _, _, off = unit_off(c)
            return pltpu.async_copy(
                scr_hbm.at[gidx.at[pl.ds(off, _BC)]],
                rows4.at[pl.ds(slot * _BC, _BC), :],
                gsem,
            )

        fire_gather(0, 0)
        fire_gather(1, 1)
        fire_gather(2, 2)

        def body(cc, carry):
            for kk in range(4):
                c = cc * 4 + kk
                f, sub, off = unit_off(c)

                @pl.when(c + 3 < _UNITS)
                def _():
                    fire_gather(c + 3, (kk + 3) % 4)

                pltpu.make_async_copy(
                    scr_hbm.at[gidx.at[pl.ds(0, _BC)]],
                    rows4.at[pl.ds(0, _BC), :],
                    gsem,
                ).wait()

                @pl.when(c >= 2)
                def _():
                    pltpu.make_async_copy(
                        outb.at[pl.ds(0, _EMBED), :],
                        out_hbm.at[0, :, pl.ds(0, _BC)],
                        wsem,
                    ).wait()

                orow = (kk % 2) * _EMBED
                for j in range(_BC // 16):
                    rowvec = iot + (kk * _BC + j * 16)
                    lb = (idxb[pl.ds(off + j * 16, 16)] & 3) * _EMBED
                    for eb in range(_EMBED // 16):
                        vals = [
                            plsc.load_gather(
                                rows4, [rowvec, lb + (eb * 16 + i)]
                            )
                            for i in range(16)
                        ]
                        for i in range(16):
                            outb.at[orow + eb * 16 + i][pl.ds(j * 16, 16)] = (
                                vals[i]
                            )
                pltpu.async_copy(
                    outb.at[pl.ds(orow, _EMBED), :],
                    out_hbm.at[f, :, pl.ds(b0w + sub * _BC, _BC)],
                    wsem,
                )
            return carry

        lax.fori_loop(0, _UNITS // 4, body, 0)
        for _ in range(2):
            pltpu.make_async_copy(
                outb.at[pl.ds(0, _EMBED), :],
                out_hbm.at[0, :, pl.ds(0, _BC)],
                wsem,
            ).wait()

    return k(idx_t, tab_t)


def kernel(tensor, table):
    idx_t = tensor.T.astype(jnp.int32)
    tab_t = table.T
    out_t, _ = _lookup(idx_t, tab_t)
    return out_t.transpose(2, 0, 1)
